# baseline (speedup 1.0000x reference)
import functools

import jax
import jax.numpy as jnp
from jax import lax
from jax.experimental import pallas as pl
from jax.experimental.pallas import tpu as pltpu
from jax.experimental.pallas import tpu_sc as plsc

ALPHA = 1.702
LIMIT = 7.0
TOP_K = 2
BT = 256
NC, NS = 2, 16
NW = NC * NS
CH = 64

_PREC = jax.lax.Precision.HIGHEST


def _router_body(x_ref, w_ref, b_ref, i0_ref, i1_ref, w0_ref, w1_ref):
    x = x_ref[...]
    logits = jnp.dot(x, w_ref[...], preferred_element_type=jnp.float32,
                     precision=_PREC) + b_ref[...]
    lane = lax.broadcasted_iota(jnp.int32, logits.shape, 1)
    m0 = jnp.max(logits, axis=1, keepdims=True)
    i0 = jnp.min(jnp.where(logits == m0, lane, 1 << 30), axis=1, keepdims=True)
    l2 = jnp.where(lane == i0, -3e38, logits)
    m1 = jnp.max(l2, axis=1, keepdims=True)
    i1 = jnp.min(jnp.where(l2 == m1, lane, 1 << 30), axis=1, keepdims=True)
    e1 = jnp.exp(m1 - m0)
    w0 = 1.0 / (1.0 + e1)
    i0_ref[...] = i0
    i1_ref[...] = i1
    w0_ref[...] = w0
    w1_ref[...] = e1 * w0


def _run_router(x, router_weight, router_bias, interpret=False):
    T, H = x.shape
    E = router_weight.shape[0]
    wt = jnp.zeros((H, 128), jnp.float32).at[:, :E].set(router_weight.T)
    bp = jnp.full((1, 128), -1e30, jnp.float32).at[0, :E].set(router_bias)
    TB = 1024
    outs = pl.pallas_call(
        _router_body,
        grid=(T // TB,),
        in_specs=[
            pl.BlockSpec((TB, H), lambda b: (b, 0)),
            pl.BlockSpec((H, 128), lambda b: (0, 0)),
            pl.BlockSpec((1, 128), lambda b: (0, 0)),
        ],
        out_specs=[
            pl.BlockSpec((TB, 1), lambda b: (b, 0)),
            pl.BlockSpec((TB, 1), lambda b: (b, 0)),
            pl.BlockSpec((TB, 1), lambda b: (b, 0)),
            pl.BlockSpec((TB, 1), lambda b: (b, 0)),
        ],
        out_shape=[
            jax.ShapeDtypeStruct((T, 1), jnp.int32),
            jax.ShapeDtypeStruct((T, 1), jnp.int32),
            jax.ShapeDtypeStruct((T, 1), jnp.float32),
            jax.ShapeDtypeStruct((T, 1), jnp.float32),
        ],
        interpret=interpret,
    )(x, wt, bp)
    return outs


def _sc_gather(src, idx, H, ch=CH):
    R = idx.shape[0]
    rows_per_w = R // NW
    n_chunks = rows_per_w // ch
    mesh = plsc.VectorSubcoreMesh(core_axis_name="c", subcore_axis_name="s",
                                  num_cores=NC, num_subcores=NS)

    def body(src_hbm, idx_hbm, out_hbm, idx_v, rows0, rows1, sem0, sem1):
        wid = lax.axis_index("s") * NC + lax.axis_index("c")
        base = wid * rows_per_w
        pltpu.sync_copy(idx_hbm.at[pl.ds(base, rows_per_w)], idx_v)
        bufs = (rows0, rows1)
        sems = (sem0, sem1)
        copies = [None] * n_chunks
        copies[0] = pltpu.async_copy(
            src_hbm.at[idx_v.at[pl.ds(0, ch)]], bufs[0], sems[0])
        for c in range(n_chunks):
            if c + 1 < n_chunks:
                copies[c + 1] = pltpu.async_copy(
                    src_hbm.at[idx_v.at[pl.ds((c + 1) * ch, ch)]],
                    bufs[(c + 1) % 2], sems[(c + 1) % 2])
            copies[c].wait()
            pltpu.sync_copy(bufs[c % 2],
                            out_hbm.at[pl.ds(base + c * ch, ch)])

    k = pl.kernel(
        body,
        out_type=jax.ShapeDtypeStruct((R, H), src.dtype),
        mesh=mesh,
        scratch_types=[
            pltpu.VMEM((rows_per_w,), jnp.int32),
            pltpu.VMEM((ch, H), src.dtype),
            pltpu.VMEM((ch, H), src.dtype),
            pltpu.SemaphoreType.DMA,
            pltpu.SemaphoreType.DMA,
        ],
    )
    return k(src, idx)


def _gemm_body(be_ref, xs_ref, gw_ref, uw_ref, gb_ref, ub_ref, dw_ref,
               db_ref, ws_ref, out_ref):
    x = xs_ref[...]
    g = jnp.dot(x, gw_ref[0], preferred_element_type=jnp.float32) + gb_ref[0]
    u = jnp.dot(x, uw_ref[0], preferred_element_type=jnp.float32) + ub_ref[0]
    g = jnp.minimum(g, LIMIT)
    u = jnp.clip(u, -LIMIT, LIMIT)
    act = (u + 1.0) * (g * jax.nn.sigmoid(g * ALPHA))
    y = jnp.dot(act.astype(jnp.bfloat16), dw_ref[0],
                preferred_element_type=jnp.float32) + db_ref[0]
    out_ref[...] = y * ws_ref[:, 0:1]


def _run_gemm(block_expert, xs, gw, uw, gb, ub, dw, db, ws_pad,
              interpret=False):
    NPAD, H = xs.shape
    I = gw.shape[2]
    NB = NPAD // BT
    grid_spec = pltpu.PrefetchScalarGridSpec(
        num_scalar_prefetch=1,
        grid=(NB,),
        in_specs=[
            pl.BlockSpec((BT, H), lambda b, be: (b, 0)),
            pl.BlockSpec((1, H, I), lambda b, be: (be[b], 0, 0)),
            pl.BlockSpec((1, H, I), lambda b, be: (be[b], 0, 0)),
            pl.BlockSpec((1, 1, I), lambda b, be: (be[b], 0, 0)),
            pl.BlockSpec((1, 1, I), lambda b, be: (be[b], 0, 0)),
            pl.BlockSpec((1, I, H), lambda b, be: (be[b], 0, 0)),
            pl.BlockSpec((1, 1, H), lambda b, be: (be[b], 0, 0)),
            pl.BlockSpec((BT, 128), lambda b, be: (b, 0)),
        ],
        out_specs=pl.BlockSpec((BT, H), lambda b, be: (b, 0)),
    )
    return pl.pallas_call(
        _gemm_body,
        grid_spec=grid_spec,
        out_shape=jax.ShapeDtypeStruct((NPAD, H), jnp.float32),
        compiler_params=pltpu.CompilerParams(
            vmem_limit_bytes=100 * 1024 * 1024),
        interpret=interpret,
    )(block_expert, xs, gw, uw, gb, ub, dw, db, ws_pad)


def _add_body(a_ref, b_ref, o_ref):
    o_ref[...] = a_ref[...] + b_ref[...]


def _run_add(yg, T, H, interpret=False):
    TB = 512
    noff = T // TB
    return pl.pallas_call(
        _add_body,
        grid=(noff,),
        in_specs=[
            pl.BlockSpec((TB, H), lambda b: (b, 0)),
            pl.BlockSpec((TB, H), lambda b: (b + noff, 0)),
        ],
        out_specs=pl.BlockSpec((TB, H), lambda b: (b, 0)),
        out_shape=jax.ShapeDtypeStruct((T, H), jnp.float32),
        interpret=interpret,
    )(yg, yg)


def _dispatch(i0, i1, w0, w1, E, T):
    N = TOP_K * T
    NB = N // BT + E
    NPAD = NB * BT
    ef = jnp.stack([i0, i1], axis=1).reshape(-1)
    wf = jnp.stack([w0, w1], axis=1).reshape(-1)
    onehot = (ef[:, None] == jnp.arange(E, dtype=jnp.int32)[None, :])
    ranks = jnp.cumsum(onehot.astype(jnp.int32), axis=0)
    counts = ranks[-1]
    pc = ((counts + BT - 1) // BT) * BT
    bstart = jnp.concatenate(
        [jnp.zeros((1,), jnp.int32), jnp.cumsum(pc)])[:E] // BT
    r = jnp.take_along_axis(ranks, ef[:, None], axis=1)[:, 0] - 1
    posn = bstart[ef] * BT + r
    tokn = jnp.arange(N, dtype=jnp.int32) // TOP_K
    tok_sorted = jnp.zeros((NPAD,), jnp.int32).at[posn].set(tokn)
    w_sorted = jnp.zeros((NPAD,), jnp.float32).at[posn].set(wf)
    block_expert = (jnp.sum(
        (jnp.arange(NB, dtype=jnp.int32)[:, None] >= bstart[None, :]),
        axis=1) - 1).astype(jnp.int32)
    pos2 = posn.reshape(T, TOP_K)
    pos_flat = jnp.concatenate([pos2[:, 0], pos2[:, 1]])
    return tok_sorted, w_sorted, block_expert, pos_flat, NPAD


def kernel(hidden_states, router_weight, router_bias, gate_up_proj,
           gate_up_proj_bias, down_proj, down_proj_bias):
    bsz, seq_len, H = hidden_states.shape
    T = bsz * seq_len
    E = router_weight.shape[0]
    x = hidden_states.reshape(T, H)

    logits = x @ router_weight.T + router_bias
    top_vals, top_idx = jax.lax.top_k(logits, TOP_K)
    top_vals = jax.nn.softmax(top_vals, axis=-1)
    tok_sorted, w_sorted, block_expert, pos_flat, NPAD = _dispatch(
        top_idx[:, 0], top_idx[:, 1], top_vals[:, 0], top_vals[:, 1], E, T)

    I2 = gate_up_proj.shape[2] // 2
    guw = jnp.transpose(gate_up_proj.reshape(E, H, I2, 2),
                        (3, 0, 1, 2)).astype(jnp.bfloat16)
    gw = guw[0]
    uw = guw[1]
    dw = down_proj.astype(jnp.bfloat16)
    gb = gate_up_proj_bias[:, 0::2][:, None, :]
    ub = gate_up_proj_bias[:, 1::2][:, None, :]
    db = down_proj_bias[:, None, :]

    xp = jax.lax.bitcast_convert_type(
        x.astype(jnp.bfloat16).reshape(T, H // 2, 2), jnp.int32)
    xsp = _sc_gather(xp, tok_sorted, H // 2)
    xs = jax.lax.bitcast_convert_type(xsp, jnp.bfloat16).reshape(NPAD, H)
    ws_pad = jnp.broadcast_to(w_sorted[:, None], (NPAD, 128))
    ys = _run_gemm(block_expert, xs, gw, uw, gb, ub, dw,
                   db, ws_pad)
    yg = _sc_gather(ys, pos_flat, H, ch=32)
    out = _run_add(yg, T, H)
    return out.reshape(bsz, seq_len, H)


# speedup vs baseline: 1.3727x; 1.3727x over previous
import functools

import jax
import jax.numpy as jnp
from jax import lax
from jax.experimental import pallas as pl
from jax.experimental.pallas import tpu as pltpu
from jax.experimental.pallas import tpu_sc as plsc

ALPHA = 1.702
LIMIT = 7.0
TOP_K = 2
BT = 256
NC, NS = 2, 16
NW = NC * NS
CH = 64

_PREC = jax.lax.Precision.HIGHEST


def _router_body(x_ref, w_ref, b_ref, i0_ref, i1_ref, w0_ref, w1_ref):
    x = x_ref[...]
    logits = jnp.dot(x, w_ref[...], preferred_element_type=jnp.float32,
                     precision=_PREC) + b_ref[...]
    lane = lax.broadcasted_iota(jnp.int32, logits.shape, 1)
    m0 = jnp.max(logits, axis=1, keepdims=True)
    i0 = jnp.min(jnp.where(logits == m0, lane, 1 << 30), axis=1, keepdims=True)
    l2 = jnp.where(lane == i0, -3e38, logits)
    m1 = jnp.max(l2, axis=1, keepdims=True)
    i1 = jnp.min(jnp.where(l2 == m1, lane, 1 << 30), axis=1, keepdims=True)
    e1 = jnp.exp(m1 - m0)
    w0 = 1.0 / (1.0 + e1)
    i0_ref[...] = i0
    i1_ref[...] = i1
    w0_ref[...] = w0
    w1_ref[...] = e1 * w0


def _run_router(x, router_weight, router_bias, interpret=False):
    T, H = x.shape
    E = router_weight.shape[0]
    wt = jnp.zeros((H, 128), jnp.float32).at[:, :E].set(router_weight.T)
    bp = jnp.full((1, 128), -1e30, jnp.float32).at[0, :E].set(router_bias)
    TB = 1024
    outs = pl.pallas_call(
        _router_body,
        grid=(T // TB,),
        in_specs=[
            pl.BlockSpec((TB, H), lambda b: (b, 0)),
            pl.BlockSpec((H, 128), lambda b: (0, 0)),
            pl.BlockSpec((1, 128), lambda b: (0, 0)),
        ],
        out_specs=[
            pl.BlockSpec((TB, 1), lambda b: (b, 0)),
            pl.BlockSpec((TB, 1), lambda b: (b, 0)),
            pl.BlockSpec((TB, 1), lambda b: (b, 0)),
            pl.BlockSpec((TB, 1), lambda b: (b, 0)),
        ],
        out_shape=[
            jax.ShapeDtypeStruct((T, 1), jnp.int32),
            jax.ShapeDtypeStruct((T, 1), jnp.int32),
            jax.ShapeDtypeStruct((T, 1), jnp.float32),
            jax.ShapeDtypeStruct((T, 1), jnp.float32),
        ],
        interpret=interpret,
    )(x, wt, bp)
    return outs


def _sc_gather(src, idx, H, ch=CH):
    R = idx.shape[0]
    rows_per_w = R // NW
    n_chunks = rows_per_w // ch
    mesh = plsc.VectorSubcoreMesh(core_axis_name="c", subcore_axis_name="s",
                                  num_cores=NC, num_subcores=NS)

    def body(src_hbm, idx_hbm, out_hbm, idx_v, rows0, rows1, sem0, sem1):
        wid = lax.axis_index("s") * NC + lax.axis_index("c")
        base = wid * rows_per_w
        pltpu.sync_copy(idx_hbm.at[pl.ds(base, rows_per_w)], idx_v)
        bufs = (rows0, rows1)
        sems = (sem0, sem1)
        copies = [None] * n_chunks
        copies[0] = pltpu.async_copy(
            src_hbm.at[idx_v.at[pl.ds(0, ch)]], bufs[0], sems[0])
        for c in range(n_chunks):
            if c + 1 < n_chunks:
                copies[c + 1] = pltpu.async_copy(
                    src_hbm.at[idx_v.at[pl.ds((c + 1) * ch, ch)]],
                    bufs[(c + 1) % 2], sems[(c + 1) % 2])
            copies[c].wait()
            pltpu.sync_copy(bufs[c % 2],
                            out_hbm.at[pl.ds(base + c * ch, ch)])

    k = pl.kernel(
        body,
        out_type=jax.ShapeDtypeStruct((R, H), src.dtype),
        mesh=mesh,
        scratch_types=[
            pltpu.VMEM((rows_per_w,), jnp.int32),
            pltpu.VMEM((ch, H), src.dtype),
            pltpu.VMEM((ch, H), src.dtype),
            pltpu.SemaphoreType.DMA,
            pltpu.SemaphoreType.DMA,
        ],
    )
    return k(src, idx)


def _gemm_body(be_ref, xs_ref, gw_ref, uw_ref, gb_ref, ub_ref, dw_ref,
               db_ref, ws_ref, out_ref):
    x = xs_ref[...].astype(jnp.bfloat16)
    g = jnp.dot(x, gw_ref[0], preferred_element_type=jnp.float32) + gb_ref[0]
    u = jnp.dot(x, uw_ref[0], preferred_element_type=jnp.float32) + ub_ref[0]
    g = jnp.minimum(g, LIMIT)
    u = jnp.clip(u, -LIMIT, LIMIT)
    act = (u + 1.0) * (g * jax.nn.sigmoid(g * ALPHA))
    y = jnp.dot(act.astype(jnp.bfloat16), dw_ref[0],
                preferred_element_type=jnp.float32) + db_ref[0]
    out_ref[...] = y * ws_ref[:, 0:1]


def _run_gemm(block_expert, xs, gw, uw, gb, ub, dw, db, ws_pad,
              interpret=False):
    NPAD, H = xs.shape
    I = gw.shape[2]
    NB = NPAD // BT
    grid_spec = pltpu.PrefetchScalarGridSpec(
        num_scalar_prefetch=1,
        grid=(NB,),
        in_specs=[
            pl.BlockSpec((BT, H), lambda b, be: (b, 0)),
            pl.BlockSpec((1, H, I), lambda b, be: (be[b], 0, 0)),
            pl.BlockSpec((1, H, I), lambda b, be: (be[b], 0, 0)),
            pl.BlockSpec((1, 1, I), lambda b, be: (be[b], 0, 0)),
            pl.BlockSpec((1, 1, I), lambda b, be: (be[b], 0, 0)),
            pl.BlockSpec((1, I, H), lambda b, be: (be[b], 0, 0)),
            pl.BlockSpec((1, 1, H), lambda b, be: (be[b], 0, 0)),
            pl.BlockSpec((BT, 128), lambda b, be: (b, 0)),
        ],
        out_specs=pl.BlockSpec((BT, H), lambda b, be: (b, 0)),
    )
    return pl.pallas_call(
        _gemm_body,
        grid_spec=grid_spec,
        out_shape=jax.ShapeDtypeStruct((NPAD, H), jnp.float32),
        compiler_params=pltpu.CompilerParams(
            vmem_limit_bytes=100 * 1024 * 1024),
        interpret=interpret,
    )(block_expert, xs, gw, uw, gb, ub, dw, db, ws_pad)


def _add_body(a_ref, b_ref, o_ref):
    o_ref[...] = a_ref[...] + b_ref[...]


def _run_add(yg, T, H, interpret=False):
    TB = 512
    noff = T // TB
    return pl.pallas_call(
        _add_body,
        grid=(noff,),
        in_specs=[
            pl.BlockSpec((TB, H), lambda b: (b, 0)),
            pl.BlockSpec((TB, H), lambda b: (b + noff, 0)),
        ],
        out_specs=pl.BlockSpec((TB, H), lambda b: (b, 0)),
        out_shape=jax.ShapeDtypeStruct((T, H), jnp.float32),
        interpret=interpret,
    )(yg, yg)


def _dispatch(i0, i1, w0, w1, E, T):
    N = TOP_K * T
    NB = N // BT + E
    NPAD = NB * BT
    ef = jnp.stack([i0, i1], axis=1).reshape(-1)
    wf = jnp.stack([w0, w1], axis=1).reshape(-1)
    onehot = (ef[:, None] == jnp.arange(E, dtype=jnp.int32)[None, :])
    ranks = jnp.cumsum(onehot.astype(jnp.int32), axis=0)
    counts = ranks[-1]
    pc = ((counts + BT - 1) // BT) * BT
    bstart = jnp.concatenate(
        [jnp.zeros((1,), jnp.int32), jnp.cumsum(pc)])[:E] // BT
    r = jnp.take_along_axis(ranks, ef[:, None], axis=1)[:, 0] - 1
    posn = bstart[ef] * BT + r
    tokn = jnp.arange(N, dtype=jnp.int32) // TOP_K
    tok_sorted = jnp.zeros((NPAD,), jnp.int32).at[posn].set(tokn)
    w_sorted = jnp.zeros((NPAD,), jnp.float32).at[posn].set(wf)
    block_expert = (jnp.sum(
        (jnp.arange(NB, dtype=jnp.int32)[:, None] >= bstart[None, :]),
        axis=1) - 1).astype(jnp.int32)
    pos2 = posn.reshape(T, TOP_K)
    pos_flat = jnp.concatenate([pos2[:, 0], pos2[:, 1]])
    return tok_sorted, w_sorted, block_expert, pos_flat, NPAD


def kernel(hidden_states, router_weight, router_bias, gate_up_proj,
           gate_up_proj_bias, down_proj, down_proj_bias):
    bsz, seq_len, H = hidden_states.shape
    T = bsz * seq_len
    E = router_weight.shape[0]
    x = hidden_states.reshape(T, H)

    logits = x @ router_weight.T + router_bias
    top_vals, top_idx = jax.lax.top_k(logits, TOP_K)
    top_vals = jax.nn.softmax(top_vals, axis=-1)
    tok_sorted, w_sorted, block_expert, pos_flat, NPAD = _dispatch(
        top_idx[:, 0], top_idx[:, 1], top_vals[:, 0], top_vals[:, 1], E, T)

    I2 = gate_up_proj.shape[2] // 2
    guw = jnp.transpose(gate_up_proj.reshape(E, H, I2, 2),
                        (3, 0, 1, 2)).astype(jnp.bfloat16)
    gw = guw[0]
    uw = guw[1]
    dw = down_proj.astype(jnp.bfloat16)
    gb = gate_up_proj_bias[:, 0::2][:, None, :]
    ub = gate_up_proj_bias[:, 1::2][:, None, :]
    db = down_proj_bias[:, None, :]

    xs = _sc_gather(x, tok_sorted, H, ch=48)
    ws_pad = jnp.broadcast_to(w_sorted[:, None], (NPAD, 128))
    ys = _run_gemm(block_expert, xs, gw, uw, gb, ub, dw,
                   db, ws_pad)
    yg = _sc_gather(ys, pos_flat, H, ch=32)
    out = _run_add(yg, T, H)
    return out.reshape(bsz, seq_len, H)


# speedup vs baseline: 1.5077x; 1.0983x over previous
import functools

import jax
import jax.numpy as jnp
from jax import lax
from jax.experimental import pallas as pl
from jax.experimental.pallas import tpu as pltpu
from jax.experimental.pallas import tpu_sc as plsc

ALPHA = 1.702
LIMIT = 7.0
TOP_K = 2
BT = 128
NC, NS = 2, 16
NW = NC * NS
CH = 64

_PREC = jax.lax.Precision.HIGHEST


def _router_body(x_ref, w_ref, b_ref, i0_ref, i1_ref, w0_ref, w1_ref):
    x = x_ref[...]
    logits = jnp.dot(x, w_ref[...], preferred_element_type=jnp.float32,
                     precision=_PREC) + b_ref[...]
    lane = lax.broadcasted_iota(jnp.int32, logits.shape, 1)
    m0 = jnp.max(logits, axis=1, keepdims=True)
    i0 = jnp.min(jnp.where(logits == m0, lane, 1 << 30), axis=1, keepdims=True)
    l2 = jnp.where(lane == i0, -3e38, logits)
    m1 = jnp.max(l2, axis=1, keepdims=True)
    i1 = jnp.min(jnp.where(l2 == m1, lane, 1 << 30), axis=1, keepdims=True)
    e1 = jnp.exp(m1 - m0)
    w0 = 1.0 / (1.0 + e1)
    i0_ref[...] = i0
    i1_ref[...] = i1
    w0_ref[...] = w0
    w1_ref[...] = e1 * w0


def _run_router(x, router_weight, router_bias, interpret=False):
    T, H = x.shape
    E = router_weight.shape[0]
    wt = jnp.zeros((H, 128), jnp.float32).at[:, :E].set(router_weight.T)
    bp = jnp.full((1, 128), -1e30, jnp.float32).at[0, :E].set(router_bias)
    TB = 1024
    outs = pl.pallas_call(
        _router_body,
        grid=(T // TB,),
        in_specs=[
            pl.BlockSpec((TB, H), lambda b: (b, 0)),
            pl.BlockSpec((H, 128), lambda b: (0, 0)),
            pl.BlockSpec((1, 128), lambda b: (0, 0)),
        ],
        out_specs=[
            pl.BlockSpec((TB, 1), lambda b: (b, 0)),
            pl.BlockSpec((TB, 1), lambda b: (b, 0)),
            pl.BlockSpec((TB, 1), lambda b: (b, 0)),
            pl.BlockSpec((TB, 1), lambda b: (b, 0)),
        ],
        out_shape=[
            jax.ShapeDtypeStruct((T, 1), jnp.int32),
            jax.ShapeDtypeStruct((T, 1), jnp.int32),
            jax.ShapeDtypeStruct((T, 1), jnp.float32),
            jax.ShapeDtypeStruct((T, 1), jnp.float32),
        ],
        interpret=interpret,
    )(x, wt, bp)
    return outs


def _sc_gather(src, idx, H, ch=CH):
    R = idx.shape[0]
    rows_per_w = R // NW
    n_chunks = rows_per_w // ch
    mesh = plsc.VectorSubcoreMesh(core_axis_name="c", subcore_axis_name="s",
                                  num_cores=NC, num_subcores=NS)

    def body(src_hbm, idx_hbm, out_hbm, idx_v, rows0, rows1, sem0, sem1):
        wid = lax.axis_index("s") * NC + lax.axis_index("c")
        base = wid * rows_per_w
        pltpu.sync_copy(idx_hbm.at[pl.ds(base, rows_per_w)], idx_v)
        bufs = (rows0, rows1)
        sems = (sem0, sem1)
        copies = [None] * n_chunks
        copies[0] = pltpu.async_copy(
            src_hbm.at[idx_v.at[pl.ds(0, ch)]], bufs[0], sems[0])
        for c in range(n_chunks):
            if c + 1 < n_chunks:
                copies[c + 1] = pltpu.async_copy(
                    src_hbm.at[idx_v.at[pl.ds((c + 1) * ch, ch)]],
                    bufs[(c + 1) % 2], sems[(c + 1) % 2])
            copies[c].wait()
            pltpu.sync_copy(bufs[c % 2],
                            out_hbm.at[pl.ds(base + c * ch, ch)])

    k = pl.kernel(
        body,
        out_type=jax.ShapeDtypeStruct((R, H), src.dtype),
        mesh=mesh,
        scratch_types=[
            pltpu.VMEM((rows_per_w,), jnp.int32),
            pltpu.VMEM((ch, H), src.dtype),
            pltpu.VMEM((ch, H), src.dtype),
            pltpu.SemaphoreType.DMA,
            pltpu.SemaphoreType.DMA,
        ],
    )
    return k(src, idx)


def _gemm_body(be_ref, xs_ref, gw_ref, uw_ref, gb_ref, ub_ref, dw_ref,
               db_ref, ws_ref, out_ref):
    x = xs_ref[...].astype(jnp.bfloat16)
    g = jnp.dot(x, gw_ref[0], preferred_element_type=jnp.float32) + gb_ref[0]
    u = jnp.dot(x, uw_ref[0], preferred_element_type=jnp.float32) + ub_ref[0]
    g = jnp.minimum(g, LIMIT)
    u = jnp.clip(u, -LIMIT, LIMIT)
    act = (u + 1.0) * (g * jax.nn.sigmoid(g * ALPHA))
    y = jnp.dot(act.astype(jnp.bfloat16), dw_ref[0],
                preferred_element_type=jnp.float32) + db_ref[0]
    out_ref[...] = y * ws_ref[:, 0:1]


def _run_gemm(block_expert, xs, gw, uw, gb, ub, dw, db, ws_pad,
              interpret=False):
    NPAD, H = xs.shape
    I = gw.shape[2]
    NB = NPAD // BT
    grid_spec = pltpu.PrefetchScalarGridSpec(
        num_scalar_prefetch=1,
        grid=(NB,),
        in_specs=[
            pl.BlockSpec((BT, H), lambda b, be: (b, 0)),
            pl.BlockSpec((1, H, I), lambda b, be: (be[b], 0, 0)),
            pl.BlockSpec((1, H, I), lambda b, be: (be[b], 0, 0)),
            pl.BlockSpec((1, 1, I), lambda b, be: (be[b], 0, 0)),
            pl.BlockSpec((1, 1, I), lambda b, be: (be[b], 0, 0)),
            pl.BlockSpec((1, I, H), lambda b, be: (be[b], 0, 0)),
            pl.BlockSpec((1, 1, H), lambda b, be: (be[b], 0, 0)),
            pl.BlockSpec((BT, 128), lambda b, be: (b, 0)),
        ],
        out_specs=pl.BlockSpec((BT, H), lambda b, be: (b, 0)),
    )
    return pl.pallas_call(
        _gemm_body,
        grid_spec=grid_spec,
        out_shape=jax.ShapeDtypeStruct((NPAD, H), jnp.float32),
        compiler_params=pltpu.CompilerParams(
            vmem_limit_bytes=100 * 1024 * 1024),
        interpret=interpret,
    )(block_expert, xs, gw, uw, gb, ub, dw, db, ws_pad)


def _add_body(a_ref, b_ref, o_ref):
    o_ref[...] = a_ref[...] + b_ref[...]


def _run_add(yg, T, H, interpret=False):
    TB = 512
    noff = T // TB
    return pl.pallas_call(
        _add_body,
        grid=(noff,),
        in_specs=[
            pl.BlockSpec((TB, H), lambda b: (b, 0)),
            pl.BlockSpec((TB, H), lambda b: (b + noff, 0)),
        ],
        out_specs=pl.BlockSpec((TB, H), lambda b: (b, 0)),
        out_shape=jax.ShapeDtypeStruct((T, H), jnp.float32),
        interpret=interpret,
    )(yg, yg)


def _dispatch(i0, i1, w0, w1, E, T):
    N = TOP_K * T
    NB = N // BT + E
    NPAD = NB * BT
    ef = jnp.stack([i0, i1], axis=1).reshape(-1)
    wf = jnp.stack([w0, w1], axis=1).reshape(-1)
    onehot = (ef[:, None] == jnp.arange(E, dtype=jnp.int32)[None, :])
    ranks = jnp.cumsum(onehot.astype(jnp.int32), axis=0)
    counts = ranks[-1]
    pc = ((counts + BT - 1) // BT) * BT
    bstart = jnp.concatenate(
        [jnp.zeros((1,), jnp.int32), jnp.cumsum(pc)])[:E] // BT
    r = jnp.take_along_axis(ranks, ef[:, None], axis=1)[:, 0] - 1
    posn = bstart[ef] * BT + r
    tokn = jnp.arange(N, dtype=jnp.int32) // TOP_K
    tok_sorted = jnp.zeros((NPAD,), jnp.int32).at[posn].set(tokn)
    w_sorted = jnp.zeros((NPAD,), jnp.float32).at[posn].set(wf)
    block_expert = (jnp.sum(
        (jnp.arange(NB, dtype=jnp.int32)[:, None] >= bstart[None, :]),
        axis=1) - 1).astype(jnp.int32)
    pos2 = posn.reshape(T, TOP_K)
    pos_flat = jnp.concatenate([pos2[:, 0], pos2[:, 1]])
    return tok_sorted, w_sorted, block_expert, pos_flat, NPAD


def kernel(hidden_states, router_weight, router_bias, gate_up_proj,
           gate_up_proj_bias, down_proj, down_proj_bias):
    bsz, seq_len, H = hidden_states.shape
    T = bsz * seq_len
    E = router_weight.shape[0]
    x = hidden_states.reshape(T, H)

    logits = x @ router_weight.T + router_bias
    top_vals, top_idx = jax.lax.top_k(logits, TOP_K)
    top_vals = jax.nn.softmax(top_vals, axis=-1)
    tok_sorted, w_sorted, block_expert, pos_flat, NPAD = _dispatch(
        top_idx[:, 0], top_idx[:, 1], top_vals[:, 0], top_vals[:, 1], E, T)

    I2 = gate_up_proj.shape[2] // 2
    guw = jnp.transpose(gate_up_proj.reshape(E, H, I2, 2),
                        (3, 0, 1, 2)).astype(jnp.bfloat16)
    gw = guw[0]
    uw = guw[1]
    dw = down_proj.astype(jnp.bfloat16)
    gb = gate_up_proj_bias[:, 0::2][:, None, :]
    ub = gate_up_proj_bias[:, 1::2][:, None, :]
    db = down_proj_bias[:, None, :]

    xs = _sc_gather(x, tok_sorted, H, ch=40)
    ws_pad = jnp.broadcast_to(w_sorted[:, None], (NPAD, 128))
    ys = _run_gemm(block_expert, xs, gw, uw, gb, ub, dw,
                   db, ws_pad)
    yg = _sc_gather(ys, pos_flat, H, ch=32)
    out = _run_add(yg, T, H)
    return out.reshape(bsz, seq_len, H)
